# baseline (speedup 1.0000x reference)
import functools
import math

import jax
import jax.numpy as jnp
from jax import lax
from jax.experimental import pallas as pl
from jax.experimental.pallas import tpu as pltpu
from jax.experimental.pallas import tpu_sc as plsc

_V = 32000
_PAD = 0
_SMOOTHING = 0.1
_CONF = 1.0 - _SMOOTHING
_EPS = _SMOOTHING / (_V - 1)
_C = (_V - 1) * _EPS * math.log(_EPS) + _CONF * math.log(_CONF)

_NC = 2
_NS = 16
_L = 16
_NW = _NC * _NS

_SC_ROWS = 1024
_RPW = _SC_ROWS // _NW
_RB = 64


def _sc_body(xflat_hbm, tgt_hbm, sums_hbm, cnts_hbm, drows_hbm,
             tgt_v, idx_v, vals_v, vec_v, cvec_v, dout_v, buf0, buf1,
             sem_g, sem0, sem1):
    n = tgt_v.shape[0] * _NW
    chunk = tgt_v.shape[0]
    wid = lax.axis_index("s") * _NC + lax.axis_index("c")

    base = wid * chunk
    pltpu.sync_copy(tgt_hbm.at[pl.ds(base, chunk)], tgt_v)
    iota = lax.iota(jnp.int32, _L)
    for j in range(chunk // _L):
        t16 = tgt_v[pl.ds(j * _L, _L)]
        rows16 = (base + j * _L) + iota
        idx_v[pl.ds(j * _L, _L)] = rows16 * _V + t16
    gcp = pltpu.async_copy(xflat_hbm.at[idx_v], vals_v, sem_g)

    r0 = (n - _SC_ROWS) + wid * _RPW
    w0 = r0 * _V

    bufs = (buf0, buf1)
    sems = (sem0, sem1)
    pltpu.async_copy(xflat_hbm.at[pl.ds(w0, _V)], buf0, sem0)
    pltpu.async_copy(xflat_hbm.at[pl.ds(w0 + _V, _V)], buf1, sem1)

    def _row_sum(buf):
        zero = jnp.zeros((_L,), jnp.float32)

        def inner(k, a):
            b = k * 256
            a = list(a)
            for j in range(16):
                a[j % 8] = a[j % 8] + buf[pl.ds(b + _L * j, _L)]
            return tuple(a)

        accs = lax.fori_loop(0, _V // 256, inner, (zero,) * 8)
        w = accs[0]
        for j in range(1, 8):
            w = w + accs[j]
        return w

    for r in range(_RPW):
        b = bufs[r % 2]
        s = sems[r % 2]
        pltpu.make_async_copy(
            xflat_hbm.at[pl.ds(w0 + r * _V, _V)], b, s).wait()
        dout_v[r] = _row_sum(b)
        if r + 2 < _RPW:
            pltpu.async_copy(
                xflat_hbm.at[pl.ds(w0 + (r + 2) * _V, _V)], b, s)

    gcp.wait()
    acc = jnp.zeros((_L,), jnp.float32)
    cnt = jnp.zeros((_L,), jnp.int32)
    for j in range(chunk // _L):
        t16 = tgt_v[pl.ds(j * _L, _L)]
        v16 = vals_v[pl.ds(j * _L, _L)]
        m = t16 != _PAD
        acc = acc + jnp.where(m, v16, jnp.float32(0.0))
        cnt = cnt + jnp.where(m, 1, 0)
    vec_v[...] = acc
    cvec_v[...] = cnt
    pltpu.sync_copy(vec_v, sums_hbm.at[wid])
    pltpu.sync_copy(cvec_v, cnts_hbm.at[wid])
    pltpu.sync_copy(dout_v, drows_hbm.at[wid])


def _sc_gather(xflat, tgt):
    n = tgt.shape[0]
    chunk = n // _NW
    mesh = plsc.VectorSubcoreMesh(core_axis_name="c", subcore_axis_name="s")
    run = functools.partial(
        pl.kernel,
        out_type=(
            jax.ShapeDtypeStruct((_NW, _L), jnp.float32),
            jax.ShapeDtypeStruct((_NW, _L), jnp.int32),
            jax.ShapeDtypeStruct((_NW, _RPW, _L), jnp.float32),
        ),
        mesh=mesh,
        scratch_types=(
            pltpu.VMEM((chunk,), jnp.int32),
            pltpu.VMEM((chunk,), jnp.int32),
            pltpu.VMEM((chunk,), jnp.float32),
            pltpu.VMEM((_L,), jnp.float32),
            pltpu.VMEM((_L,), jnp.int32),
            pltpu.VMEM((_RPW, _L), jnp.float32),
            pltpu.VMEM((_V,), jnp.float32),
            pltpu.VMEM((_V,), jnp.float32),
            pltpu.SemaphoreType.DMA,
            pltpu.SemaphoreType.DMA,
            pltpu.SemaphoreType.DMA,
        ),
    )(_sc_body)
    return run(xflat, tgt)


def _tc_body(t_ref, x_ref, out_ref, acc_ref):
    i = pl.program_id(0)
    mask = (t_ref[...] != _PAD).astype(jnp.float32)

    @pl.when(i == 0)
    def _init():
        acc_ref[...] = x_ref[...] * mask

    @pl.when(i > 0)
    def _acc():
        acc_ref[...] += x_ref[...] * mask

    @pl.when(i == pl.num_programs(0) - 1)
    def _fini():
        out_ref[0, 0] = jnp.sum(acc_ref[...])


def _tc_reduce(x, tgt2d):
    n, v = x.shape
    grid = ((n - _SC_ROWS) // _RB,)
    return pl.pallas_call(
        _tc_body,
        grid=grid,
        in_specs=[
            pl.BlockSpec((_RB, 1), lambda i: (i, 0)),
            pl.BlockSpec((_RB, v), lambda i: (i, 0)),
        ],
        out_specs=pl.BlockSpec((1, 1), lambda i: (0, 0),
                               memory_space=pltpu.SMEM),
        out_shape=jax.ShapeDtypeStruct((1, 1), jnp.float32),
        scratch_shapes=[pltpu.VMEM((_RB, v), jnp.float32)],
        compiler_params=pltpu.CompilerParams(
            dimension_semantics=("arbitrary",),
        ),
    )(tgt2d, x)


def _comb_body(tcp_ref, sums_ref, cnts_ref, drows_ref, tsc_ref, out_ref):
    s_t = jnp.sum(sums_ref[...])
    nv = jnp.sum(cnts_ref[...]).astype(jnp.float32)
    dmask = (tsc_ref[...] != _PAD).astype(jnp.float32)
    s_all = tcp_ref[0, 0] + jnp.sum(drows_ref[...] * dmask)
    out_ref[0, 0] = (nv * jnp.float32(_C)
                     - jnp.float32(_EPS) * s_all
                     - jnp.float32(_CONF - _EPS) * s_t) / nv


def _combine(tcp, sums, cnts, drows, tsc):
    return pl.pallas_call(
        _comb_body,
        in_specs=[
            pl.BlockSpec(memory_space=pltpu.SMEM),
            pl.BlockSpec(),
            pl.BlockSpec(),
            pl.BlockSpec(),
            pl.BlockSpec(),
        ],
        out_specs=pl.BlockSpec(memory_space=pltpu.SMEM),
        out_shape=jax.ShapeDtypeStruct((1, 1), jnp.float32),
    )(tcp, sums, cnts, drows, tsc)


def kernel(x, target):
    n, v = x.shape
    target = target.astype(jnp.int32)
    xflat = x.reshape(n * v)
    sums, cnts, drows = _sc_gather(xflat, target)
    tcp = _tc_reduce(x, target.reshape(n, 1))
    out = _combine(tcp, sums, cnts,
                   drows.reshape(_SC_ROWS, _L),
                   target[n - _SC_ROWS:].reshape(_SC_ROWS, 1))
    return out[0, 0]
